# Initial kernel scaffold; baseline (speedup 1.0000x reference)
import jax
import jax.numpy as jnp
from jax.experimental import pallas as pl


def kernel(videos, label):
    raise NotImplementedError("write your pallas kernel here")


# speedup vs baseline: 1.0006x; 1.0006x over previous
import functools

import jax
import jax.numpy as jnp
import numpy as np
from jax.experimental import pallas as pl
from jax.experimental.pallas import tpu as pltpu

CROP = 112
BETA = 0.5
EPS = 1e-08
GK = (int(0.1 * CROP) // 2) * 2 + 1
SIGMA = GK / 3.0


def _gauss1d():
    x = jnp.arange(GK, dtype=jnp.float32) - GK // 2
    g = jnp.exp(-(x ** 2) / (2.0 * SIGMA ** 2))
    return g / g.sum()


def _blur(x):
    k1 = _gauss1d()
    pad = GK // 2
    xp = jnp.pad(x, ((0, 0), (0, 0), (pad, pad), (pad, pad)), mode='reflect')
    dn = ('NCHW', 'OIHW', 'NCHW')
    y = jax.lax.conv_general_dilated(xp, k1.reshape(1, 1, GK, 1), (1, 1), 'VALID', dimension_numbers=dn)
    y = jax.lax.conv_general_dilated(y, k1.reshape(1, 1, 1, GK), (1, 1), 'VALID', dimension_numbers=dn)
    return y


def _rgb_to_hsv(img):
    r, g, b = img[:, 0], img[:, 1], img[:, 2]
    maxc = jnp.maximum(jnp.maximum(r, g), b)
    minc = jnp.minimum(jnp.minimum(r, g), b)
    v = maxc
    deltac = maxc - minc
    s = deltac / (maxc + 1e-08)
    dc = jnp.where(deltac == 0, 1.0, deltac)
    rc = (maxc - r) / dc
    gc = (maxc - g) / dc
    bc = (maxc - b) / dc
    h = jnp.where(maxc == r, bc - gc, jnp.where(maxc == g, 2.0 + rc - bc, 4.0 + gc - rc))
    h = (h / 6.0) % 1.0
    h = h * (2.0 * np.pi)
    return jnp.stack([h, s, v], axis=1)


def _norm_batch(m):
    B, H, W = m.shape
    f = m.reshape(B, -1)
    f = f - jnp.min(f, axis=-1, keepdims=True)
    f = f / (jnp.max(f, axis=-1, keepdims=True) + EPS)
    return f.reshape(B, H, W)


def _bincount(x, max_value):
    B = x.shape[0]
    tgt = jnp.zeros((B, max_value), dtype=x.dtype)
    return tgt.at[jnp.arange(B)[:, None], x].add(jnp.ones_like(x))


def _get_seg(mask, video_clips):
    B, C, T, H, W = video_clips.shape
    vc = jnp.mean(video_clips, axis=2)
    img_hsv = _rgb_to_hsv(vc.reshape(-1, C, H, W))
    mflat = mask.reshape(B, -1)
    fg_idx = jax.lax.top_k(mflat, int(0.5 * H * W))[1]
    bg_idx = jax.lax.top_k(-mflat, int(0.1 * H * W))[1]
    dimH, dimS, dimV = 10, 10, 10
    img_hsv = img_hsv.reshape(B, -1, H, W)
    img_h, img_s, img_v = img_hsv[:, 0], img_hsv[:, 1], img_hsv[:, 2]
    hx = (img_s * jnp.cos(img_h * 2.0 * np.pi) + 1.0) / 2.0
    hy = (img_s * jnp.sin(img_h * 2.0 * np.pi) + 1.0) / 2.0
    h = jnp.round(hx * (dimH - 1) + 1)
    s = jnp.round(hy * (dimS - 1) + 1)
    v = jnp.round(img_v * (dimV - 1) + 1)
    color_map = h + (s - 1) * dimH + (v - 1) * dimH * dimS
    color_map = color_map.reshape(B, -1).astype(jnp.int32)
    col_fg = jnp.take_along_axis(color_map, fg_idx, axis=-1)
    col_bg = jnp.take_along_axis(color_map, bg_idx, axis=-1)
    dict_fg = _bincount(col_fg, dimH * dimS * dimV).astype(jnp.float32)
    dict_bg = _bincount(col_bg, dimH * dimS * dimV).astype(jnp.float32) + 1.0
    dict_fg = dict_fg / (jnp.sum(dict_fg, axis=-1, keepdims=True) + EPS)
    dict_bg = dict_bg / (jnp.sum(dict_bg, axis=-1, keepdims=True) + EPS)
    pr_fg = jnp.take_along_axis(dict_fg, color_map, axis=1)
    pr_bg = jnp.take_along_axis(dict_bg, color_map, axis=1)
    refine = pr_fg / (pr_bg + pr_fg)
    m = _blur(refine.reshape(-1, 1, H, W))
    m = _norm_batch(m.reshape(-1, H, W))
    num_fg = int(BETA * H * W)
    sidx = jax.lax.top_k(m.reshape(B, -1), num_fg)[1]
    mask0 = jnp.zeros((B, H * W), dtype=m.dtype)
    b_index = jnp.repeat(jnp.arange(B), num_fg)
    mask0 = mask0.at[b_index, sidx.reshape(-1)].set(1.0)
    return mask0.reshape(B, H, W)


def _get_mask(video_clips):
    B, C, T, H, W = video_clips.shape
    im_diff = jnp.mean(jnp.sum(jnp.abs(video_clips[:, :, 0:-1] - video_clips[:, :, 1:]), axis=1), axis=1)
    m = _blur(im_diff.reshape(-1, 1, H, W))
    m = _norm_batch(m.reshape(-1, H, W))
    return _get_seg(m, video_clips)


def _get_mask_per_frame(video_clips):
    B, C, T, H, W = video_clips.shape
    masks = []
    for i in range(0, T, 2):
        im_diff = jnp.sum(jnp.abs(video_clips[:, :, i] - video_clips[:, :, i + 1]), axis=1)
        m = _blur(im_diff.reshape(-1, 1, H, W))
        m = _norm_batch(m.reshape(-1, H, W))
        masks.append(_get_seg(m, video_clips))
    return masks


def _avg_pool16(x):
    H, W = x.shape[-2], x.shape[-1]
    xr = x.reshape(x.shape[:-2] + (H // 16, 16, W // 16, 16))
    return jnp.mean(xr, axis=(-3, -1))


def _fuse_body(perm_ref, vidp_ref, vid_ref, mask_ref, out_ref):
    m = mask_ref[0]
    out_ref[0] = vidp_ref[0] * (1.0 - m) + vid_ref[0] * m


def _fuse(videos, mask, perm):
    B, C, T, H, W = videos.shape
    grid_spec = pltpu.PrefetchScalarGridSpec(
        num_scalar_prefetch=1,
        grid=(B,),
        in_specs=[
            pl.BlockSpec((1, C, T, H, W), lambda b, perm_ref: (perm_ref[b], 0, 0, 0, 0)),
            pl.BlockSpec((1, C, T, H, W), lambda b, perm_ref: (b, 0, 0, 0, 0)),
            pl.BlockSpec((1, H, W), lambda b, perm_ref: (b, 0, 0)),
        ],
        out_specs=pl.BlockSpec((1, C, T, H, W), lambda b, perm_ref: (b, 0, 0, 0, 0)),
    )
    return pl.pallas_call(
        _fuse_body,
        grid_spec=grid_spec,
        out_shape=jax.ShapeDtypeStruct((B, C, T, H, W), videos.dtype),
    )(perm, videos, videos, mask)


def kernel(videos, label):
    B, C, T, H, W = videos.shape
    std = jnp.array([0.229, 0.224, 0.225], dtype=videos.dtype).reshape(1, 3, 1, 1, 1)
    mean = jnp.array([0.485, 0.456, 0.406], dtype=videos.dtype).reshape(1, 3, 1, 1, 1)
    tmp_video = videos * std + mean
    mask = _get_mask(tmp_video)
    masks_per_frame = jnp.stack(_get_mask_per_frame(tmp_video)).transpose(1, 0, 2, 3)
    index = jax.random.permutation(jax.random.key(1234), B).astype(jnp.int32)
    all_videos = _fuse(videos, mask, index)
    mask_out = _avg_pool16(mask).reshape(B, -1)
    mpf_out = _avg_pool16(masks_per_frame).reshape(B, -1)
    return (all_videos, label, (mask_out, mpf_out))


# speedup vs baseline: 14.1298x; 14.1214x over previous
import functools

import jax
import jax.numpy as jnp
import numpy as np
from jax.experimental import pallas as pl
from jax.experimental.pallas import tpu as pltpu

CROP = 112
BETA = 0.5
EPS = 1e-08
GK = (int(0.1 * CROP) // 2) * 2 + 1
SIGMA = GK / 3.0
HW = CROP * CROP
K_FG = int(0.5 * HW)
K_BG = int(0.1 * HW)
K_FINAL = int(BETA * HW)
NBINS = 1000


def _np_blur_matrix():
    x = np.arange(GK, dtype=np.float64) - GK // 2
    g = np.exp(-(x ** 2) / (2.0 * SIGMA ** 2))
    g = (g / g.sum()).astype(np.float32)
    n = CROP
    KB = np.zeros((n, n), dtype=np.float32)
    for i in range(n):
        for k in range(GK):
            src = i - GK // 2 + k
            if src < 0:
                src = -src
            elif src > n - 1:
                src = 2 * (n - 1) - src
            KB[i, src] += g[k]
    return KB


_KB = _np_blur_matrix()
_UT = np.triu(np.ones((CROP, CROP), dtype=np.float32))
_LS = np.tril(np.ones((CROP, CROP), dtype=np.float32), k=-1)
_PP = np.zeros((CROP, 7), dtype=np.float32)
for _i in range(CROP):
    _PP[_i, _i // 16] = 1.0


def _dotT(a, b):
    return jax.lax.dot_general(a, b, (((0,), (0,)), ((), ())),
                               preferred_element_type=jnp.float32)


def _gauss_np():
    x = np.arange(GK, dtype=np.float32) - GK // 2
    g = np.exp(-(x ** 2) / (2.0 * SIGMA ** 2)).astype(np.float32)
    g = (g / g.sum()).astype(np.float32)
    return np.asarray(g, dtype=np.float32).astype("bfloat16").astype(np.float32)


_GB = [float(w) for w in _gauss_np()]


def _bf16(x):
    return x.astype(jnp.bfloat16).astype(jnp.float32)


def _blur_norm(x, kb, kbt):
    top = [x[5 - p:6 - p, :] for p in range(5)]
    bot = [x[110 - q:111 - q, :] for q in range(5)]
    xp = _bf16(jnp.concatenate(top + [x] + bot, axis=0))
    y = _GB[0] * xp[0:CROP, :]
    for k in range(1, GK):
        y = y + _GB[k] * xp[k:k + CROP, :]
    left = [y[:, 5 - p:6 - p] for p in range(5)]
    right = [y[:, 110 - q:111 - q] for q in range(5)]
    yp = _bf16(jnp.concatenate(left + [y] + right, axis=1))
    z = _GB[0] * yp[:, 0:CROP]
    for k in range(1, GK):
        z = z + _GB[k] * yp[:, k:k + CROP]
    mn = jnp.min(z)
    f = z - mn
    mx = jnp.max(f)
    return f / (mx + EPS)


def _flat_rank(eq, ut, ls):
    eqf = eq.astype(jnp.float32)
    rowcum = jnp.dot(eqf, ut, preferred_element_type=jnp.float32)
    rowtot = rowcum[:, CROP - 1:CROP]
    offs = jnp.dot(ls, rowtot, preferred_element_type=jnp.float32)
    return rowcum + offs


def _topk_weights(m2d, k, largest, ut, ls):
    keys = jax.lax.bitcast_convert_type(m2d, jnp.int32)
    kk = jnp.int32(k)
    top = jnp.int32(0x3F800000)

    if largest:
        def body(_, lohi):
            lo, hi = lohi
            mid = (lo + hi) >> 1
            cnt = jnp.sum((keys >= mid).astype(jnp.int32))
            ok = cnt >= kk
            return jnp.where(ok, mid, lo), jnp.where(ok, hi, mid)

        lo, _ = jax.lax.fori_loop(0, 31, body, (jnp.int32(0), top))
        t = lo
        strict = keys > t
    else:
        def body(_, lohi):
            lo, hi = lohi
            mid = (lo + hi) >> 1
            cnt = jnp.sum((keys <= mid).astype(jnp.int32))
            ok = cnt >= kk
            return jnp.where(ok, lo, mid), jnp.where(ok, mid, hi)

        _, hi = jax.lax.fori_loop(0, 31, body, (jnp.int32(-1), top))
        t = hi
        strict = keys < t

    cnt_strict = jnp.sum(strict.astype(jnp.int32))
    r = (kk - cnt_strict).astype(jnp.float32)
    eq = keys == t
    rank = _flat_rank(eq, ut, ls)
    sel = strict | (eq & (rank <= r))
    return sel.astype(jnp.float32)


def _seg_body(vid_ref, kb_ref, kbt_ref, ut_ref, ls_ref, pp_ref,
              mask_ref, pools_ref):
    kb = kb_ref[...]
    kbt = kbt_ref[...]
    ut = ut_ref[...]
    ls = ls_ref[...]
    pp = pp_ref[...]

    stds = (0.229, 0.224, 0.225)
    means = (0.485, 0.456, 0.406)

    vcs = []
    dsum = jnp.zeros((CROP, CROP), jnp.float32)
    dpair = [jnp.zeros((CROP, CROP), jnp.float32) for _ in range(8)]
    for c in range(3):
        tc = vid_ref[0, c] * stds[c] + means[c]
        vcs.append(jnp.mean(tc, axis=0))
        ad = jnp.abs(tc[:-1] - tc[1:])
        dsum = dsum + jnp.sum(ad, axis=0)
        for j in range(8):
            dpair[j] = dpair[j] + ad[2 * j]

    r, g, b = vcs
    maxc = jnp.maximum(jnp.maximum(r, g), b)
    minc = jnp.minimum(jnp.minimum(r, g), b)
    vv = maxc
    deltac = maxc - minc
    s = deltac / (maxc + 1e-08)
    dc = jnp.where(deltac == 0, 1.0, deltac)
    rc = (maxc - r) / dc
    gc = (maxc - g) / dc
    bc = (maxc - b) / dc
    h = jnp.where(maxc == r, bc - gc,
                  jnp.where(maxc == g, 2.0 + rc - bc, 4.0 + gc - rc))
    h6 = h / 6.0
    hmod = h6 - jnp.floor(h6)
    hang = hmod * (2.0 * np.pi)
    hx = (s * jnp.cos(hang * 2.0 * np.pi) + 1.0) / 2.0
    hy = (s * jnp.sin(hang * 2.0 * np.pi) + 1.0) / 2.0
    hq = jnp.round(hx * 9.0 + 1.0)
    sq = jnp.round(hy * 9.0 + 1.0)
    vq = jnp.round(vv * 9.0 + 1.0)
    cm = (hq + (sq - 1.0) * 10.0 + (vq - 1.0) * 100.0).astype(jnp.int32)
    c0 = cm - 1
    amap = c0 >> 3
    bmap = c0 & 7

    aoh = (amap[:, :, None] ==
           jax.lax.broadcasted_iota(jnp.int32, (CROP, CROP, 128), 2)
           ).astype(jnp.float32)
    aohm = aoh.reshape(HW, 128)
    boh = (bmap[:, :, None] ==
           jax.lax.broadcasted_iota(jnp.int32, (CROP, CROP, 8), 2)
           ).astype(jnp.float32)

    maps = [dsum / 15.0] + dpair

    for mi in range(9):
        m = _blur_norm(maps[mi], kb, kbt)
        wfg = _topk_weights(m, K_FG, True, ut, ls)
        wbg = _topk_weights(m, K_BG, False, ut, ls)

        lhs3 = jnp.concatenate(
            [boh * wfg[:, :, None], boh * wbg[:, :, None]], axis=2)
        lhs = lhs3.reshape(HW, 16)
        hist = _dotT(lhs, aohm)
        hf = hist[0:8, :]
        hb = hist[8:16, :]
        df = hf / 6272.0
        db = (hb + 1.0) / 2254.0
        val = df / (df + db)

        refine = jnp.zeros((CROP, CROP), jnp.float32)
        for bd in range(8):
            tab = jnp.broadcast_to(val[bd:bd + 1, :], (CROP, 128))
            gb = jnp.take_along_axis(tab, amap, axis=1)
            refine = refine + jnp.where(bmap == bd, gb, 0.0)

        m2 = _blur_norm(refine, kb, kbt)
        msk = _topk_weights(m2, K_FINAL, True, ut, ls)

        if mi == 0:
            mask_ref[0] = msk
        pool = jnp.dot(_dotT(pp, msk), pp,
                       preferred_element_type=jnp.float32) / 256.0
        pools_ref[0, mi, 0:7, 0:7] = pool


def _seg(videos):
    B, C, T, H, W = videos.shape
    kb = jnp.asarray(_KB)
    kbt = jnp.asarray(_KB.T.copy())
    ut = jnp.asarray(_UT)
    ls = jnp.asarray(_LS)
    pp = jnp.asarray(_PP)
    c2 = lambda bb: (0, 0)
    return pl.pallas_call(
        _seg_body,
        grid=(B,),
        in_specs=[
            pl.BlockSpec((1, C, T, H, W), lambda bb: (bb, 0, 0, 0, 0)),
            pl.BlockSpec((H, W), c2),
            pl.BlockSpec((H, W), c2),
            pl.BlockSpec((H, W), c2),
            pl.BlockSpec((H, W), c2),
            pl.BlockSpec((H, 7), c2),
        ],
        out_specs=[
            pl.BlockSpec((1, H, W), lambda bb: (bb, 0, 0)),
            pl.BlockSpec((1, 9, 8, 128), lambda bb: (bb, 0, 0, 0)),
        ],
        out_shape=[
            jax.ShapeDtypeStruct((B, H, W), jnp.float32),
            jax.ShapeDtypeStruct((B, 9, 8, 128), jnp.float32),
        ],
    )(videos, kb, kbt, ut, ls, pp)


def _fuse_body(perm_ref, vidp_ref, vid_ref, mask_ref, out_ref):
    m = mask_ref[0]
    out_ref[0] = vidp_ref[0] * (1.0 - m) + vid_ref[0] * m


def _fuse(videos, mask, perm):
    B, C, T, H, W = videos.shape
    grid_spec = pltpu.PrefetchScalarGridSpec(
        num_scalar_prefetch=1,
        grid=(B,),
        in_specs=[
            pl.BlockSpec((1, C, T, H, W), lambda b, p: (p[b], 0, 0, 0, 0)),
            pl.BlockSpec((1, C, T, H, W), lambda b, p: (b, 0, 0, 0, 0)),
            pl.BlockSpec((1, H, W), lambda b, p: (b, 0, 0)),
        ],
        out_specs=pl.BlockSpec((1, C, T, H, W), lambda b, p: (b, 0, 0, 0, 0)),
    )
    return pl.pallas_call(
        _fuse_body,
        grid_spec=grid_spec,
        out_shape=jax.ShapeDtypeStruct((B, C, T, H, W), videos.dtype),
    )(perm, videos, videos, mask)


def kernel(videos, label):
    B, C, T, H, W = videos.shape
    mask, pools = _seg(videos)
    index = jax.random.permutation(jax.random.key(1234), B).astype(jnp.int32)
    all_videos = _fuse(videos, mask, index)
    mask_out = pools[:, 0, :7, :7].reshape(B, 49)
    mpf_out = pools[:, 1:9, :7, :7].reshape(B, 392)
    return (all_videos, label, (mask_out, mpf_out))


# speedup vs baseline: 29.1598x; 2.0637x over previous
import functools

import jax
import jax.numpy as jnp
import numpy as np
from jax.experimental import pallas as pl
from jax.experimental.pallas import tpu as pltpu

CROP = 112
BETA = 0.5
EPS = 1e-08
GK = (int(0.1 * CROP) // 2) * 2 + 1
SIGMA = GK / 3.0
HW = CROP * CROP
K_FG = int(0.5 * HW)
K_BG = int(0.1 * HW)
K_FINAL = int(BETA * HW)
NBINS = 1000


def _np_blur_matrix():
    x = np.arange(GK, dtype=np.float64) - GK // 2
    g = np.exp(-(x ** 2) / (2.0 * SIGMA ** 2))
    g = (g / g.sum()).astype(np.float32)
    n = CROP
    KB = np.zeros((n, n), dtype=np.float32)
    for i in range(n):
        for k in range(GK):
            src = i - GK // 2 + k
            if src < 0:
                src = -src
            elif src > n - 1:
                src = 2 * (n - 1) - src
            KB[i, src] += g[k]
    return KB


_KB = _np_blur_matrix()
_UT = np.triu(np.ones((CROP, CROP), dtype=np.float32))
_LS = np.tril(np.ones((CROP, CROP), dtype=np.float32), k=-1)
_PP = np.zeros((CROP, 7), dtype=np.float32)
for _i in range(CROP):
    _PP[_i, _i // 16] = 1.0


def _dotT(a, b):
    return jax.lax.dot_general(a, b, (((0,), (0,)), ((), ())),
                               preferred_element_type=jnp.float32)


def _gauss_np():
    x = np.arange(GK, dtype=np.float32) - GK // 2
    g = np.exp(-(x ** 2) / (2.0 * SIGMA ** 2)).astype(np.float32)
    g = (g / g.sum()).astype(np.float32)
    return np.asarray(g, dtype=np.float32).astype("bfloat16").astype(np.float32)


_GB = [float(w) for w in _gauss_np()]


def _bf16(x):
    return x.astype(jnp.bfloat16).astype(jnp.float32)


def _blur_norm(x, kb, kbt):
    top = [x[5 - p:6 - p, :] for p in range(5)]
    bot = [x[110 - q:111 - q, :] for q in range(5)]
    xp = _bf16(jnp.concatenate(top + [x] + bot, axis=0))
    y = _GB[0] * xp[0:CROP, :]
    for k in range(1, GK):
        y = y + _GB[k] * xp[k:k + CROP, :]
    left = [y[:, 5 - p:6 - p] for p in range(5)]
    right = [y[:, 110 - q:111 - q] for q in range(5)]
    yp = _bf16(jnp.concatenate(left + [y] + right, axis=1))
    z = _GB[0] * yp[:, 0:CROP]
    for k in range(1, GK):
        z = z + _GB[k] * yp[:, k:k + CROP]
    mn = jnp.min(z)
    f = z - mn
    mx = jnp.max(f)
    return f / (mx + EPS)


def _flat_rank(eq, ut, ls):
    eqf = eq.astype(jnp.float32)
    rowcum = jnp.dot(eqf, ut, preferred_element_type=jnp.float32)
    rowtot = rowcum[:, CROP - 1:CROP]
    offs = jnp.dot(ls, rowtot, preferred_element_type=jnp.float32)
    return rowcum + offs


_TOPBITS = 0x3F800000


def _joint_bisect(specs):
    los, his = [], []
    for keys, k, largest in specs:
        if largest:
            los.append(jnp.int32(0))
            his.append(jnp.int32(_TOPBITS))
        else:
            los.append(jnp.int32(-1))
            his.append(jnp.int32(_TOPBITS))

    def body(_, carry):
        los, his = carry
        nlos, nhis = [], []
        for (keys, k, largest), lo, hi in zip(specs, los, his):
            mid = (lo + hi) >> 1
            if largest:
                cnt = jnp.sum((keys >= mid).astype(jnp.int32))
                ok = cnt >= k
                nlos.append(jnp.where(ok, mid, lo))
                nhis.append(jnp.where(ok, hi, mid))
            else:
                cnt = jnp.sum((keys <= mid).astype(jnp.int32))
                ok = cnt >= k
                nlos.append(jnp.where(ok, lo, mid))
                nhis.append(jnp.where(ok, mid, hi))
        return nlos, nhis

    los, his = jax.lax.fori_loop(0, 31, body, (los, his))
    return [lo if largest else hi
            for (keys, k, largest), lo, hi in zip(specs, los, his)]


def _weights_from_threshold(keys, t, k, largest, ut, ls):
    strict = (keys > t) if largest else (keys < t)
    cnt_strict = jnp.sum(strict.astype(jnp.int32))
    r = (jnp.int32(k) - cnt_strict).astype(jnp.float32)
    eq = keys == t
    rank = _flat_rank(eq, ut, ls)
    sel = strict | (eq & (rank <= r))
    return sel.astype(jnp.float32)


def _seg_body(vid_ref, kb_ref, kbt_ref, ut_ref, ls_ref, pp_ref,
              mask_ref, pools_ref):
    kb = kb_ref[...]
    kbt = kbt_ref[...]
    ut = ut_ref[...]
    ls = ls_ref[...]
    pp = pp_ref[...]

    stds = (0.229, 0.224, 0.225)
    means = (0.485, 0.456, 0.406)

    vcs = []
    dsum = jnp.zeros((CROP, CROP), jnp.float32)
    dpair = [jnp.zeros((CROP, CROP), jnp.float32) for _ in range(8)]
    for c in range(3):
        tc = vid_ref[0, c] * stds[c] + means[c]
        vcs.append(jnp.mean(tc, axis=0))
        ad = jnp.abs(tc[:-1] - tc[1:])
        dsum = dsum + jnp.sum(ad, axis=0)
        for j in range(8):
            dpair[j] = dpair[j] + ad[2 * j]

    r, g, b = vcs
    maxc = jnp.maximum(jnp.maximum(r, g), b)
    minc = jnp.minimum(jnp.minimum(r, g), b)
    vv = maxc
    deltac = maxc - minc
    s = deltac / (maxc + 1e-08)
    dc = jnp.where(deltac == 0, 1.0, deltac)
    rc = (maxc - r) / dc
    gc = (maxc - g) / dc
    bc = (maxc - b) / dc
    h = jnp.where(maxc == r, bc - gc,
                  jnp.where(maxc == g, 2.0 + rc - bc, 4.0 + gc - rc))
    h6 = h / 6.0
    hmod = h6 - jnp.floor(h6)
    hang = hmod * (2.0 * np.pi)
    hx = (s * jnp.cos(hang * 2.0 * np.pi) + 1.0) / 2.0
    hy = (s * jnp.sin(hang * 2.0 * np.pi) + 1.0) / 2.0
    hq = jnp.round(hx * 9.0 + 1.0)
    sq = jnp.round(hy * 9.0 + 1.0)
    vq = jnp.round(vv * 9.0 + 1.0)
    cm = (hq + (sq - 1.0) * 10.0 + (vq - 1.0) * 100.0).astype(jnp.int32)
    c0 = cm - 1
    amap = c0 >> 3
    bmap = c0 & 7

    aoh = (amap[:, :, None] ==
           jax.lax.broadcasted_iota(jnp.int32, (CROP, CROP, 128), 2)
           ).astype(jnp.float32)
    aohm = aoh.reshape(HW, 128)
    io128 = jax.lax.broadcasted_iota(jnp.int32, (CROP, CROP, 128), 2)
    bm3 = bmap[:, :, None]

    maps = [dsum / 15.0] + dpair

    ms = [_blur_norm(maps[mi], kb, kbt) for mi in range(9)]
    keys = [jax.lax.bitcast_convert_type(m, jnp.int32) for m in ms]

    specs = []
    for mi in range(9):
        specs.append((keys[mi], K_FG, True))
        specs.append((keys[mi], K_BG, False))
    ths = _joint_bisect(specs)

    m2s = []
    for mi in range(9):
        wfg = _weights_from_threshold(keys[mi], ths[2 * mi], K_FG, True, ut, ls)
        wbg = _weights_from_threshold(keys[mi], ths[2 * mi + 1], K_BG, False, ut, ls)

        lhs3 = jnp.where(io128 == bm3, wfg[:, :, None],
                         jnp.where(io128 == bm3 + 8, wbg[:, :, None], 0.0))
        hist = _dotT(lhs3.reshape(HW, 128), aohm)
        hf = hist[0:8, :]
        hb = hist[8:16, :]
        df = hf / 6272.0
        db = (hb + 1.0) / 2254.0
        val = df / (df + db)

        refine = jnp.zeros((CROP, CROP), jnp.float32)
        for bd in range(8):
            tab = jnp.broadcast_to(val[bd:bd + 1, :], (CROP, 128))
            gb = jnp.take_along_axis(tab, amap, axis=1)
            refine = refine + jnp.where(bmap == bd, gb, 0.0)

        m2s.append(_blur_norm(refine, kb, kbt))

    keys2 = [jax.lax.bitcast_convert_type(m2, jnp.int32) for m2 in m2s]
    ths2 = _joint_bisect([(k2, K_FINAL, True) for k2 in keys2])

    for mi in range(9):
        msk = _weights_from_threshold(keys2[mi], ths2[mi], K_FINAL, True, ut, ls)
        if mi == 0:
            mask_ref[0] = msk
        pool = jnp.dot(_dotT(pp, msk), pp,
                       preferred_element_type=jnp.float32) / 256.0
        pools_ref[0, mi, 0:7, 0:7] = pool


def _seg(videos):
    B, C, T, H, W = videos.shape
    kb = jnp.asarray(_KB)
    kbt = jnp.asarray(_KB.T.copy())
    ut = jnp.asarray(_UT)
    ls = jnp.asarray(_LS)
    pp = jnp.asarray(_PP)
    c2 = lambda bb: (0, 0)
    return pl.pallas_call(
        _seg_body,
        grid=(B,),
        in_specs=[
            pl.BlockSpec((1, C, T, H, W), lambda bb: (bb, 0, 0, 0, 0)),
            pl.BlockSpec((H, W), c2),
            pl.BlockSpec((H, W), c2),
            pl.BlockSpec((H, W), c2),
            pl.BlockSpec((H, W), c2),
            pl.BlockSpec((H, 7), c2),
        ],
        out_specs=[
            pl.BlockSpec((1, H, W), lambda bb: (bb, 0, 0)),
            pl.BlockSpec((1, 9, 8, 128), lambda bb: (bb, 0, 0, 0)),
        ],
        out_shape=[
            jax.ShapeDtypeStruct((B, H, W), jnp.float32),
            jax.ShapeDtypeStruct((B, 9, 8, 128), jnp.float32),
        ],
    )(videos, kb, kbt, ut, ls, pp)


def _fuse_body(perm_ref, vidp_ref, vid_ref, mask_ref, out_ref):
    m = mask_ref[0]
    out_ref[0] = vidp_ref[0] * (1.0 - m) + vid_ref[0] * m


def _fuse(videos, mask, perm):
    B, C, T, H, W = videos.shape
    grid_spec = pltpu.PrefetchScalarGridSpec(
        num_scalar_prefetch=1,
        grid=(B,),
        in_specs=[
            pl.BlockSpec((1, C, T, H, W), lambda b, p: (p[b], 0, 0, 0, 0)),
            pl.BlockSpec((1, C, T, H, W), lambda b, p: (b, 0, 0, 0, 0)),
            pl.BlockSpec((1, H, W), lambda b, p: (b, 0, 0)),
        ],
        out_specs=pl.BlockSpec((1, C, T, H, W), lambda b, p: (b, 0, 0, 0, 0)),
    )
    return pl.pallas_call(
        _fuse_body,
        grid_spec=grid_spec,
        out_shape=jax.ShapeDtypeStruct((B, C, T, H, W), videos.dtype),
    )(perm, videos, videos, mask)


def kernel(videos, label):
    B, C, T, H, W = videos.shape
    mask, pools = _seg(videos)
    index = jax.random.permutation(jax.random.key(1234), B).astype(jnp.int32)
    all_videos = _fuse(videos, mask, index)
    mask_out = pools[:, 0, :7, :7].reshape(B, 49)
    mpf_out = pools[:, 1:9, :7, :7].reshape(B, 392)
    return (all_videos, label, (mask_out, mpf_out))


# speedup vs baseline: 66.9643x; 2.2965x over previous
import functools

import jax
import jax.numpy as jnp
import numpy as np
from jax import lax
from jax.experimental import pallas as pl
from jax.experimental.pallas import tpu as pltpu
from jax.experimental.pallas import tpu_sc as plsc

CROP = 112
BETA = 0.5
EPS = 1e-08
GK = (int(0.1 * CROP) // 2) * 2 + 1
SIGMA = GK / 3.0
HW = CROP * CROP
K_FG = int(0.5 * HW)
K_BG = int(0.1 * HW)
K_FINAL = int(BETA * HW)
NBINS = 1000


def _np_blur_matrix():
    x = np.arange(GK, dtype=np.float64) - GK // 2
    g = np.exp(-(x ** 2) / (2.0 * SIGMA ** 2))
    g = (g / g.sum()).astype(np.float32)
    n = CROP
    KB = np.zeros((n, n), dtype=np.float32)
    for i in range(n):
        for k in range(GK):
            src = i - GK // 2 + k
            if src < 0:
                src = -src
            elif src > n - 1:
                src = 2 * (n - 1) - src
            KB[i, src] += g[k]
    return KB


_KB = _np_blur_matrix()
_UT = np.triu(np.ones((CROP, CROP), dtype=np.float32))
_LS = np.tril(np.ones((CROP, CROP), dtype=np.float32), k=-1)
_PP = np.zeros((CROP, 7), dtype=np.float32)
for _i in range(CROP):
    _PP[_i, _i // 16] = 1.0


def _dotT(a, b):
    return jax.lax.dot_general(a, b, (((0,), (0,)), ((), ())),
                               preferred_element_type=jnp.float32)


def _gauss_np():
    x = np.arange(GK, dtype=np.float32) - GK // 2
    g = np.exp(-(x ** 2) / (2.0 * SIGMA ** 2)).astype(np.float32)
    g = (g / g.sum()).astype(np.float32)
    return np.asarray(g, dtype=np.float32).astype("bfloat16").astype(np.float32)


_GB = [float(w) for w in _gauss_np()]


def _bf16(x):
    return x.astype(jnp.bfloat16).astype(jnp.float32)


def _blur_norm(x):
    top = [x[5 - p:6 - p, :] for p in range(5)]
    bot = [x[110 - q:111 - q, :] for q in range(5)]
    xp = _bf16(jnp.concatenate(top + [x] + bot, axis=0))
    y = _GB[0] * xp[0:CROP, :]
    for k in range(1, GK):
        y = y + _GB[k] * xp[k:k + CROP, :]
    left = [y[:, 5 - p:6 - p] for p in range(5)]
    right = [y[:, 110 - q:111 - q] for q in range(5)]
    yp = _bf16(jnp.concatenate(left + [y] + right, axis=1))
    z = _GB[0] * yp[:, 0:CROP]
    for k in range(1, GK):
        z = z + _GB[k] * yp[:, k:k + CROP]
    mn = jnp.min(z)
    f = z - mn
    mx = jnp.max(f)
    return f / (mx + EPS)


def _flat_rank(eq, ut, ls):
    eqf = eq.astype(jnp.float32)
    rowcum = jnp.dot(eqf, ut, preferred_element_type=jnp.float32)
    rowtot = rowcum[:, CROP - 1:CROP]
    offs = jnp.dot(ls, rowtot, preferred_element_type=jnp.float32)
    return rowcum + offs


_TOPBITS = 0x3F800000


def _joint_bisect(specs):
    los, his = [], []
    for keys, k, largest in specs:
        if largest:
            los.append(jnp.int32(0))
            his.append(jnp.int32(_TOPBITS))
        else:
            los.append(jnp.int32(-1))
            his.append(jnp.int32(_TOPBITS))

    def body(_, carry):
        los, his = carry
        nlos, nhis = [], []
        for (keys, k, largest), lo, hi in zip(specs, los, his):
            mid = (lo + hi) >> 1
            if largest:
                cnt = jnp.sum((keys >= mid).astype(jnp.int32))
                ok = cnt >= k
                nlos.append(jnp.where(ok, mid, lo))
                nhis.append(jnp.where(ok, hi, mid))
            else:
                cnt = jnp.sum((keys <= mid).astype(jnp.int32))
                ok = cnt >= k
                nlos.append(jnp.where(ok, lo, mid))
                nhis.append(jnp.where(ok, mid, hi))
        return nlos, nhis

    los, his = jax.lax.fori_loop(0, 31, body, (los, his))
    return [lo if largest else hi
            for (keys, k, largest), lo, hi in zip(specs, los, his)]


def _weights_from_threshold(keys, t, k, largest, ut, ls):
    strict = (keys > t) if largest else (keys < t)
    cnt_strict = jnp.sum(strict.astype(jnp.int32))
    r = (jnp.int32(k) - cnt_strict).astype(jnp.float32)
    eq = keys == t
    rank = _flat_rank(eq, ut, ls)
    return strict | (eq & (rank <= r))


def _seg_a_body(vid_ref, ut_ref, ls_ref, c0_ref, wcode_ref):
    ut = ut_ref[...]
    ls = ls_ref[...]

    stds = (0.229, 0.224, 0.225)
    means = (0.485, 0.456, 0.406)

    vcs = []
    dsum = jnp.zeros((CROP, CROP), jnp.float32)
    dpair = [jnp.zeros((CROP, CROP), jnp.float32) for _ in range(8)]
    for c in range(3):
        tc = vid_ref[0, c] * stds[c] + means[c]
        vcs.append(jnp.mean(tc, axis=0))
        ad = jnp.abs(tc[:-1] - tc[1:])
        dsum = dsum + jnp.sum(ad, axis=0)
        for j in range(8):
            dpair[j] = dpair[j] + ad[2 * j]

    r, g, b = vcs
    maxc = jnp.maximum(jnp.maximum(r, g), b)
    minc = jnp.minimum(jnp.minimum(r, g), b)
    vv = maxc
    deltac = maxc - minc
    s = deltac / (maxc + 1e-08)
    dc = jnp.where(deltac == 0, 1.0, deltac)
    rc = (maxc - r) / dc
    gc = (maxc - g) / dc
    bc = (maxc - b) / dc
    h = jnp.where(maxc == r, bc - gc,
                  jnp.where(maxc == g, 2.0 + rc - bc, 4.0 + gc - rc))
    h6 = h / 6.0
    hmod = h6 - jnp.floor(h6)
    hang = hmod * (2.0 * np.pi)
    hx = (s * jnp.cos(hang * 2.0 * np.pi) + 1.0) / 2.0
    hy = (s * jnp.sin(hang * 2.0 * np.pi) + 1.0) / 2.0
    hq = jnp.round(hx * 9.0 + 1.0)
    sq = jnp.round(hy * 9.0 + 1.0)
    vq = jnp.round(vv * 9.0 + 1.0)
    cm = (hq + (sq - 1.0) * 10.0 + (vq - 1.0) * 100.0).astype(jnp.int32)
    c0 = cm - 1
    c0_ref[0] = c0

    maps = [dsum / 15.0] + dpair

    ms = [_blur_norm(maps[mi]) for mi in range(9)]
    keys = [jax.lax.bitcast_convert_type(m, jnp.int32) for m in ms]

    specs = []
    for mi in range(9):
        specs.append((keys[mi], K_FG, True))
        specs.append((keys[mi], K_BG, False))
    ths = _joint_bisect(specs)

    for mi in range(9):
        sfg = _weights_from_threshold(keys[mi], ths[2 * mi], K_FG, True, ut, ls)
        sbg = _weights_from_threshold(keys[mi], ths[2 * mi + 1], K_BG, False, ut, ls)
        wcode_ref[0, mi] = (sfg.astype(jnp.int32)
                            | (sbg.astype(jnp.int32) << 13))


def _seg_a(videos):
    B, C, T, H, W = videos.shape
    ut = jnp.asarray(_UT)
    ls = jnp.asarray(_LS)
    c2 = lambda bb: (0, 0)
    return pl.pallas_call(
        _seg_a_body,
        grid=(B,),
        in_specs=[
            pl.BlockSpec((1, C, T, H, W), lambda bb: (bb, 0, 0, 0, 0)),
            pl.BlockSpec((H, W), c2),
            pl.BlockSpec((H, W), c2),
        ],
        out_specs=[
            pl.BlockSpec((1, H, W), lambda bb: (bb, 0, 0)),
            pl.BlockSpec((1, 9, H, W), lambda bb: (bb, 0, 0, 0)),
        ],
        out_shape=[
            jax.ShapeDtypeStruct((B, H, W), jnp.int32),
            jax.ShapeDtypeStruct((B, 9, H, W), jnp.int32),
        ],
    )(videos, ut, ls)


_NBINP = 1008


def _sc_refine(c0f, wcf):
    B = c0f.shape[0] // HW
    mesh = plsc.VectorSubcoreMesh(core_axis_name="c", subcore_axis_name="s")

    @functools.partial(
        pl.kernel, mesh=mesh,
        out_type=jax.ShapeDtypeStruct((B * 9 * HW,), jnp.float32),
        compiler_params=pltpu.CompilerParams(needs_layout_passes=False),
        scratch_types=[
            pltpu.VMEM((HW,), jnp.int32),
            pltpu.VMEM((HW,), jnp.int32),
            pltpu.VMEM((16 * _NBINP,), jnp.int32),
            pltpu.VMEM((_NBINP,), jnp.float32),
            pltpu.VMEM((HW,), jnp.float32),
        ],
    )
    def k(c0_hbm, wc_hbm, out_hbm, c0_v, wc_v, h16_v, val_v, ref_v):
        wid = lax.axis_index("s") * 2 + lax.axis_index("c")
        pltpu.sync_copy(c0_hbm.at[pl.ds(wid * HW, HW)], c0_v)
        io16 = lax.broadcasted_iota(jnp.int32, (16,), 0)
        lane_off = io16 * _NBINP
        zz = jnp.zeros((16,), jnp.int32)
        for mi in range(9):
            pltpu.sync_copy(wc_hbm.at[pl.ds((wid * 9 + mi) * HW, HW)], wc_v)

            def zb(i, _):
                h16_v[pl.ds(i * 16, 16)] = zz
                return 0

            lax.fori_loop(0, _NBINP, zb, 0)

            def sb(i, _):
                idx = c0_v[pl.ds(i * 16, 16)]
                code = wc_v[pl.ds(i * 16, 16)]
                plsc.addupdate_scatter(h16_v, [idx + lane_off], code)
                return 0

            lax.fori_loop(0, HW // 16, sb, 0)

            def vb(i, _):
                acc = h16_v[pl.ds(i * 16, 16)]
                for l in range(1, 16):
                    acc = acc + h16_v[pl.ds(l * _NBINP + i * 16, 16)]
                hf = (acc & 0x1FFF).astype(jnp.float32)
                hb = (acc >> 13).astype(jnp.float32)
                df = hf / 6272.0
                db = (hb + 1.0) / 2254.0
                val_v[pl.ds(i * 16, 16)] = df / (df + db)
                return 0

            lax.fori_loop(0, _NBINP // 16, vb, 0)

            def gb(i, _):
                idx = c0_v[pl.ds(i * 16, 16)]
                ref_v[pl.ds(i * 16, 16)] = plsc.load_gather(val_v, [idx])
                return 0

            lax.fori_loop(0, HW // 16, gb, 0)
            pltpu.sync_copy(ref_v, out_hbm.at[pl.ds((wid * 9 + mi) * HW, HW)])

    return k(c0f, wcf)


def _seg_c_body(refine_ref, ut_ref, ls_ref, pp_ref, mask_ref, pools_ref):
    ut = ut_ref[...]
    ls = ls_ref[...]
    pp = pp_ref[...]

    m2s = [_blur_norm(refine_ref[0, mi]) for mi in range(9)]
    keys2 = [jax.lax.bitcast_convert_type(m2, jnp.int32) for m2 in m2s]
    ths2 = _joint_bisect([(k2, K_FINAL, True) for k2 in keys2])

    for mi in range(9):
        msk = _weights_from_threshold(
            keys2[mi], ths2[mi], K_FINAL, True, ut, ls).astype(jnp.float32)
        if mi == 0:
            mask_ref[0] = msk
        pool = jnp.dot(_dotT(pp, msk), pp,
                       preferred_element_type=jnp.float32) / 256.0
        pools_ref[0, mi, 0:7, 0:7] = pool


def _seg_c(refine):
    B = refine.shape[0]
    H = W = CROP
    ut = jnp.asarray(_UT)
    ls = jnp.asarray(_LS)
    pp = jnp.asarray(_PP)
    c2 = lambda bb: (0, 0)
    return pl.pallas_call(
        _seg_c_body,
        grid=(B,),
        in_specs=[
            pl.BlockSpec((1, 9, H, W), lambda bb: (bb, 0, 0, 0)),
            pl.BlockSpec((H, W), c2),
            pl.BlockSpec((H, W), c2),
            pl.BlockSpec((H, 7), c2),
        ],
        out_specs=[
            pl.BlockSpec((1, H, W), lambda bb: (bb, 0, 0)),
            pl.BlockSpec((1, 9, 8, 128), lambda bb: (bb, 0, 0, 0)),
        ],
        out_shape=[
            jax.ShapeDtypeStruct((B, H, W), jnp.float32),
            jax.ShapeDtypeStruct((B, 9, 8, 128), jnp.float32),
        ],
    )(refine, ut, ls, pp)


def _fuse_body(perm_ref, vidp_ref, vid_ref, mask_ref, out_ref):
    m = mask_ref[0]
    out_ref[0] = vidp_ref[0] * (1.0 - m) + vid_ref[0] * m


def _fuse(videos, mask, perm):
    B, C, T, H, W = videos.shape
    grid_spec = pltpu.PrefetchScalarGridSpec(
        num_scalar_prefetch=1,
        grid=(B,),
        in_specs=[
            pl.BlockSpec((1, C, T, H, W), lambda b, p: (p[b], 0, 0, 0, 0)),
            pl.BlockSpec((1, C, T, H, W), lambda b, p: (b, 0, 0, 0, 0)),
            pl.BlockSpec((1, H, W), lambda b, p: (b, 0, 0)),
        ],
        out_specs=pl.BlockSpec((1, C, T, H, W), lambda b, p: (b, 0, 0, 0, 0)),
    )
    return pl.pallas_call(
        _fuse_body,
        grid_spec=grid_spec,
        out_shape=jax.ShapeDtypeStruct((B, C, T, H, W), videos.dtype),
    )(perm, videos, videos, mask)


def kernel(videos, label):
    B, C, T, H, W = videos.shape
    c0, wcode = _seg_a(videos)
    refine = _sc_refine(c0.reshape(B * HW), wcode.reshape(B * 9 * HW))
    mask, pools = _seg_c(refine.reshape(B, 9, H, W))
    index = jax.random.permutation(jax.random.key(1234), B).astype(jnp.int32)
    all_videos = _fuse(videos, mask, index)
    mask_out = pools[:, 0, :7, :7].reshape(B, 49)
    mpf_out = pools[:, 1:9, :7, :7].reshape(B, 392)
    return (all_videos, label, (mask_out, mpf_out))


# speedup vs baseline: 67.6467x; 1.0102x over previous
import functools

import jax
import jax.numpy as jnp
import numpy as np
from jax import lax
from jax.experimental import pallas as pl
from jax.experimental.pallas import tpu as pltpu
from jax.experimental.pallas import tpu_sc as plsc

CROP = 112
BETA = 0.5
EPS = 1e-08
GK = (int(0.1 * CROP) // 2) * 2 + 1
SIGMA = GK / 3.0
HW = CROP * CROP
K_FG = int(0.5 * HW)
K_BG = int(0.1 * HW)
K_FINAL = int(BETA * HW)
NBINS = 1000


def _np_blur_matrix():
    x = np.arange(GK, dtype=np.float64) - GK // 2
    g = np.exp(-(x ** 2) / (2.0 * SIGMA ** 2))
    g = (g / g.sum()).astype(np.float32)
    n = CROP
    KB = np.zeros((n, n), dtype=np.float32)
    for i in range(n):
        for k in range(GK):
            src = i - GK // 2 + k
            if src < 0:
                src = -src
            elif src > n - 1:
                src = 2 * (n - 1) - src
            KB[i, src] += g[k]
    return KB


_KB = _np_blur_matrix()
_UT = np.triu(np.ones((CROP, CROP), dtype=np.float32))
_LS = np.tril(np.ones((CROP, CROP), dtype=np.float32), k=-1)
_PP = np.zeros((CROP, 7), dtype=np.float32)
for _i in range(CROP):
    _PP[_i, _i // 16] = 1.0


def _dotT(a, b):
    return jax.lax.dot_general(a, b, (((0,), (0,)), ((), ())),
                               preferred_element_type=jnp.float32)


def _gauss_np():
    x = np.arange(GK, dtype=np.float32) - GK // 2
    g = np.exp(-(x ** 2) / (2.0 * SIGMA ** 2)).astype(np.float32)
    g = (g / g.sum()).astype(np.float32)
    return np.asarray(g, dtype=np.float32).astype("bfloat16").astype(np.float32)


_GB = [float(w) for w in _gauss_np()]


def _bf16(x):
    return x.astype(jnp.bfloat16).astype(jnp.float32)


def _blur_norm(x):
    top = [x[5 - p:6 - p, :] for p in range(5)]
    bot = [x[110 - q:111 - q, :] for q in range(5)]
    xp = _bf16(jnp.concatenate(top + [x] + bot, axis=0))
    y = _GB[0] * xp[0:CROP, :]
    for k in range(1, GK):
        y = y + _GB[k] * xp[k:k + CROP, :]
    left = [y[:, 5 - p:6 - p] for p in range(5)]
    right = [y[:, 110 - q:111 - q] for q in range(5)]
    yp = _bf16(jnp.concatenate(left + [y] + right, axis=1))
    z = _GB[0] * yp[:, 0:CROP]
    for k in range(1, GK):
        z = z + _GB[k] * yp[:, k:k + CROP]
    mn = jnp.min(jnp.min(z, axis=1, keepdims=True), axis=0, keepdims=True)
    f = z - mn
    mx = jnp.max(jnp.max(f, axis=1, keepdims=True), axis=0, keepdims=True)
    return f / (mx + EPS)


def _flat_rank(eq, ut, ls):
    eqf = eq.astype(jnp.float32)
    rowcum = jnp.dot(eqf, ut, preferred_element_type=jnp.float32)
    rowtot = rowcum[:, CROP - 1:CROP]
    offs = jnp.dot(ls, rowtot, preferred_element_type=jnp.float32)
    return rowcum + offs


_TOPBITS = 0x3F800000


def _joint_bisect(specs):
    los, his = [], []
    for keys, k, largest in specs:
        if largest:
            los.append(jnp.int32(0))
            his.append(jnp.int32(_TOPBITS))
        else:
            los.append(jnp.int32(-1))
            his.append(jnp.int32(_TOPBITS))

    def body(_, carry):
        los, his = carry
        nlos, nhis = [], []
        for (keys, k, largest), lo, hi in zip(specs, los, his):
            mid = (lo + hi) >> 1
            if largest:
                cnt = jnp.sum((keys >= mid).astype(jnp.int32))
                ok = cnt >= k
                nlos.append(jnp.where(ok, mid, lo))
                nhis.append(jnp.where(ok, hi, mid))
            else:
                cnt = jnp.sum((keys <= mid).astype(jnp.int32))
                ok = cnt >= k
                nlos.append(jnp.where(ok, lo, mid))
                nhis.append(jnp.where(ok, mid, hi))
        return nlos, nhis

    los, his = jax.lax.fori_loop(0, 31, body, (los, his))
    return [lo if largest else hi
            for (keys, k, largest), lo, hi in zip(specs, los, his)]


def _weights_from_threshold(keys, t, k, largest, ut, ls):
    strict = (keys > t) if largest else (keys < t)
    cnt_strict = jnp.sum(strict.astype(jnp.int32))
    r = (jnp.int32(k) - cnt_strict).astype(jnp.float32)
    eq = keys == t
    rank = _flat_rank(eq, ut, ls)
    return strict | (eq & (rank <= r))


def _seg_a_body(vid_ref, ut_ref, ls_ref, c0_ref, wcode_ref):
    ut = ut_ref[...]
    ls = ls_ref[...]

    stds = (0.229, 0.224, 0.225)
    means = (0.485, 0.456, 0.406)

    vcs = []
    dsum = jnp.zeros((CROP, CROP), jnp.float32)
    dpair = [jnp.zeros((CROP, CROP), jnp.float32) for _ in range(8)]
    for c in range(3):
        tc = vid_ref[0, c] * stds[c] + means[c]
        vcs.append(jnp.mean(tc, axis=0))
        ad = jnp.abs(tc[:-1] - tc[1:])
        dsum = dsum + jnp.sum(ad, axis=0)
        for j in range(8):
            dpair[j] = dpair[j] + ad[2 * j]

    r, g, b = vcs
    maxc = jnp.maximum(jnp.maximum(r, g), b)
    minc = jnp.minimum(jnp.minimum(r, g), b)
    vv = maxc
    deltac = maxc - minc
    s = deltac / (maxc + 1e-08)
    dc = jnp.where(deltac == 0, 1.0, deltac)
    rc = (maxc - r) / dc
    gc = (maxc - g) / dc
    bc = (maxc - b) / dc
    h = jnp.where(maxc == r, bc - gc,
                  jnp.where(maxc == g, 2.0 + rc - bc, 4.0 + gc - rc))
    h6 = h / 6.0
    hmod = h6 - jnp.floor(h6)
    hang = hmod * (2.0 * np.pi)
    hx = (s * jnp.cos(hang * 2.0 * np.pi) + 1.0) / 2.0
    hy = (s * jnp.sin(hang * 2.0 * np.pi) + 1.0) / 2.0
    hq = jnp.round(hx * 9.0 + 1.0)
    sq = jnp.round(hy * 9.0 + 1.0)
    vq = jnp.round(vv * 9.0 + 1.0)
    cm = (hq + (sq - 1.0) * 10.0 + (vq - 1.0) * 100.0).astype(jnp.int32)
    c0 = cm - 1
    c0_ref[0] = c0

    maps = [dsum / 15.0] + dpair

    ms = [_blur_norm(maps[mi]) for mi in range(9)]
    keys = [jax.lax.bitcast_convert_type(m, jnp.int32) for m in ms]

    specs = []
    for mi in range(9):
        specs.append((keys[mi], K_FG, True))
        specs.append((keys[mi], K_BG, False))
    ths = _joint_bisect(specs)

    for mi in range(9):
        sfg = _weights_from_threshold(keys[mi], ths[2 * mi], K_FG, True, ut, ls)
        sbg = _weights_from_threshold(keys[mi], ths[2 * mi + 1], K_BG, False, ut, ls)
        wcode_ref[0, mi] = (sfg.astype(jnp.int32)
                            | (sbg.astype(jnp.int32) << 13))


def _seg_a(videos):
    B, C, T, H, W = videos.shape
    ut = jnp.asarray(_UT)
    ls = jnp.asarray(_LS)
    c2 = lambda bb: (0, 0)
    return pl.pallas_call(
        _seg_a_body,
        grid=(B,),
        in_specs=[
            pl.BlockSpec((1, C, T, H, W), lambda bb: (bb, 0, 0, 0, 0)),
            pl.BlockSpec((H, W), c2),
            pl.BlockSpec((H, W), c2),
        ],
        out_specs=[
            pl.BlockSpec((1, H, W), lambda bb: (bb, 0, 0)),
            pl.BlockSpec((1, 9, H, W), lambda bb: (bb, 0, 0, 0)),
        ],
        out_shape=[
            jax.ShapeDtypeStruct((B, H, W), jnp.int32),
            jax.ShapeDtypeStruct((B, 9, H, W), jnp.int32),
        ],
    )(videos, ut, ls)


_NBINP = 1008


def _sc_refine(c0f, wcf):
    B = c0f.shape[0] // HW
    mesh = plsc.VectorSubcoreMesh(core_axis_name="c", subcore_axis_name="s")

    @functools.partial(
        pl.kernel, mesh=mesh,
        out_type=jax.ShapeDtypeStruct((B * 9 * HW,), jnp.float32),
        compiler_params=pltpu.CompilerParams(needs_layout_passes=False),
        scratch_types=[
            pltpu.VMEM((HW,), jnp.int32),
            pltpu.VMEM((HW,), jnp.int32),
            pltpu.VMEM((16 * _NBINP,), jnp.int32),
            pltpu.VMEM((_NBINP,), jnp.float32),
            pltpu.VMEM((HW,), jnp.float32),
        ],
    )
    def k(c0_hbm, wc_hbm, out_hbm, c0_v, wc_v, h16_v, val_v, ref_v):
        wid = lax.axis_index("s") * 2 + lax.axis_index("c")
        pltpu.sync_copy(c0_hbm.at[pl.ds(wid * HW, HW)], c0_v)
        io16 = lax.broadcasted_iota(jnp.int32, (16,), 0)
        lane_off = io16 * _NBINP
        zz = jnp.zeros((16,), jnp.int32)
        for mi in range(9):
            pltpu.sync_copy(wc_hbm.at[pl.ds((wid * 9 + mi) * HW, HW)], wc_v)

            def zb(i, _):
                h16_v[pl.ds(i * 16, 16)] = zz
                return 0

            lax.fori_loop(0, _NBINP, zb, 0)

            def sb(i, _):
                idx = c0_v[pl.ds(i * 16, 16)]
                code = wc_v[pl.ds(i * 16, 16)]
                plsc.addupdate_scatter(h16_v, [idx + lane_off], code)
                return 0

            lax.fori_loop(0, HW // 16, sb, 0)

            def vb(i, _):
                acc = h16_v[pl.ds(i * 16, 16)]
                for l in range(1, 16):
                    acc = acc + h16_v[pl.ds(l * _NBINP + i * 16, 16)]
                hf = (acc & 0x1FFF).astype(jnp.float32)
                hb = (acc >> 13).astype(jnp.float32)
                df = hf / 6272.0
                db = (hb + 1.0) / 2254.0
                val_v[pl.ds(i * 16, 16)] = df / (df + db)
                return 0

            lax.fori_loop(0, _NBINP // 16, vb, 0)

            def gb(i, _):
                idx = c0_v[pl.ds(i * 16, 16)]
                ref_v[pl.ds(i * 16, 16)] = plsc.load_gather(val_v, [idx])
                return 0

            lax.fori_loop(0, HW // 16, gb, 0)
            pltpu.sync_copy(ref_v, out_hbm.at[pl.ds((wid * 9 + mi) * HW, HW)])

    return k(c0f, wcf)


def _seg_c_body(refine_ref, ut_ref, ls_ref, pp_ref, mask_ref, pools_ref):
    ut = ut_ref[...]
    ls = ls_ref[...]
    pp = pp_ref[...]

    m2s = [_blur_norm(refine_ref[0, mi]) for mi in range(9)]
    keys2 = [jax.lax.bitcast_convert_type(m2, jnp.int32) for m2 in m2s]
    ths2 = _joint_bisect([(k2, K_FINAL, True) for k2 in keys2])

    for mi in range(9):
        msk = _weights_from_threshold(
            keys2[mi], ths2[mi], K_FINAL, True, ut, ls).astype(jnp.float32)
        if mi == 0:
            mask_ref[0] = msk
        pool = jnp.dot(_dotT(pp, msk), pp,
                       preferred_element_type=jnp.float32) / 256.0
        pools_ref[0, mi, 0:7, 0:7] = pool


def _seg_c(refine):
    B = refine.shape[0]
    H = W = CROP
    ut = jnp.asarray(_UT)
    ls = jnp.asarray(_LS)
    pp = jnp.asarray(_PP)
    c2 = lambda bb: (0, 0)
    return pl.pallas_call(
        _seg_c_body,
        grid=(B,),
        in_specs=[
            pl.BlockSpec((1, 9, H, W), lambda bb: (bb, 0, 0, 0)),
            pl.BlockSpec((H, W), c2),
            pl.BlockSpec((H, W), c2),
            pl.BlockSpec((H, 7), c2),
        ],
        out_specs=[
            pl.BlockSpec((1, H, W), lambda bb: (bb, 0, 0)),
            pl.BlockSpec((1, 9, 8, 128), lambda bb: (bb, 0, 0, 0)),
        ],
        out_shape=[
            jax.ShapeDtypeStruct((B, H, W), jnp.float32),
            jax.ShapeDtypeStruct((B, 9, 8, 128), jnp.float32),
        ],
    )(refine, ut, ls, pp)


def _fuse_body(perm_ref, vidp_ref, vid_ref, mask_ref, out_ref):
    m = mask_ref[0]
    out_ref[0] = vidp_ref[0] * (1.0 - m) + vid_ref[0] * m


def _fuse(videos, mask, perm):
    B, C, T, H, W = videos.shape
    grid_spec = pltpu.PrefetchScalarGridSpec(
        num_scalar_prefetch=1,
        grid=(B,),
        in_specs=[
            pl.BlockSpec((1, C, T, H, W), lambda b, p: (p[b], 0, 0, 0, 0)),
            pl.BlockSpec((1, C, T, H, W), lambda b, p: (b, 0, 0, 0, 0)),
            pl.BlockSpec((1, H, W), lambda b, p: (b, 0, 0)),
        ],
        out_specs=pl.BlockSpec((1, C, T, H, W), lambda b, p: (b, 0, 0, 0, 0)),
    )
    return pl.pallas_call(
        _fuse_body,
        grid_spec=grid_spec,
        out_shape=jax.ShapeDtypeStruct((B, C, T, H, W), videos.dtype),
    )(perm, videos, videos, mask)


def kernel(videos, label):
    B, C, T, H, W = videos.shape
    c0, wcode = _seg_a(videos)
    refine = _sc_refine(c0.reshape(B * HW), wcode.reshape(B * 9 * HW))
    mask, pools = _seg_c(refine.reshape(B, 9, H, W))
    index = jax.random.permutation(jax.random.key(1234), B).astype(jnp.int32)
    all_videos = _fuse(videos, mask, index)
    mask_out = pools[:, 0, :7, :7].reshape(B, 49)
    mpf_out = pools[:, 1:9, :7, :7].reshape(B, 392)
    return (all_videos, label, (mask_out, mpf_out))


# speedup vs baseline: 71.3076x; 1.0541x over previous
import functools

import jax
import jax.numpy as jnp
import numpy as np
from jax import lax
from jax.experimental import pallas as pl
from jax.experimental.pallas import tpu as pltpu
from jax.experimental.pallas import tpu_sc as plsc

CROP = 112
BETA = 0.5
EPS = 1e-08
GK = (int(0.1 * CROP) // 2) * 2 + 1
SIGMA = GK / 3.0
HW = CROP * CROP
K_FG = int(0.5 * HW)
K_BG = int(0.1 * HW)
K_FINAL = int(BETA * HW)
NBINS = 1000


def _np_blur_matrix():
    x = np.arange(GK, dtype=np.float64) - GK // 2
    g = np.exp(-(x ** 2) / (2.0 * SIGMA ** 2))
    g = (g / g.sum()).astype(np.float32)
    n = CROP
    KB = np.zeros((n, n), dtype=np.float32)
    for i in range(n):
        for k in range(GK):
            src = i - GK // 2 + k
            if src < 0:
                src = -src
            elif src > n - 1:
                src = 2 * (n - 1) - src
            KB[i, src] += g[k]
    return KB


_KB = _np_blur_matrix()
_UT = np.triu(np.ones((CROP, CROP), dtype=np.float32))
_LS = np.tril(np.ones((CROP, CROP), dtype=np.float32), k=-1)
_PP = np.zeros((CROP, 7), dtype=np.float32)
for _i in range(CROP):
    _PP[_i, _i // 16] = 1.0


def _dotT(a, b):
    return jax.lax.dot_general(a, b, (((0,), (0,)), ((), ())),
                               preferred_element_type=jnp.float32)


def _gauss_np():
    x = np.arange(GK, dtype=np.float32) - GK // 2
    g = np.exp(-(x ** 2) / (2.0 * SIGMA ** 2)).astype(np.float32)
    g = (g / g.sum()).astype(np.float32)
    return np.asarray(g, dtype=np.float32).astype("bfloat16").astype(np.float32)


_GB = [float(w) for w in _gauss_np()]


def _bf16(x):
    return x.astype(jnp.bfloat16).astype(jnp.float32)


def _blur_norm(x):
    top = [x[5 - p:6 - p, :] for p in range(5)]
    bot = [x[110 - q:111 - q, :] for q in range(5)]
    xp = _bf16(jnp.concatenate(top + [x] + bot, axis=0))
    y = _GB[0] * xp[0:CROP, :]
    for k in range(1, GK):
        y = y + _GB[k] * xp[k:k + CROP, :]
    left = [y[:, 5 - p:6 - p] for p in range(5)]
    right = [y[:, 110 - q:111 - q] for q in range(5)]
    yp = _bf16(jnp.concatenate(left + [y] + right, axis=1))
    z = _GB[0] * yp[:, 0:CROP]
    for k in range(1, GK):
        z = z + _GB[k] * yp[:, k:k + CROP]
    mn = jnp.min(jnp.min(z, axis=1, keepdims=True), axis=0, keepdims=True)
    f = z - mn
    mx = jnp.max(jnp.max(f, axis=1, keepdims=True), axis=0, keepdims=True)
    return f / (mx + EPS)


def _flat_rank(eq, ut, ls):
    eqf = eq.astype(jnp.float32)
    rowcum = jnp.dot(eqf, ut, preferred_element_type=jnp.float32)
    rowtot = rowcum[:, CROP - 1:CROP]
    offs = jnp.dot(ls, rowtot, preferred_element_type=jnp.float32)
    return rowcum + offs


_TOPBITS = 0x3F800000


def _joint_bisect(specs):
    los, his = [], []
    for keys, k, largest in specs:
        if largest:
            los.append(jnp.int32(0))
            his.append(jnp.int32(_TOPBITS))
        else:
            los.append(jnp.int32(-1))
            his.append(jnp.int32(_TOPBITS))

    def body(_, carry):
        los, his = carry
        nlos, nhis = [], []
        for (keys, k, largest), lo, hi in zip(specs, los, his):
            mid = (lo + hi) >> 1
            if largest:
                cnt = jnp.sum((keys >= mid).astype(jnp.int32))
                ok = cnt >= k
                nlos.append(jnp.where(ok, mid, lo))
                nhis.append(jnp.where(ok, hi, mid))
            else:
                cnt = jnp.sum((keys <= mid).astype(jnp.int32))
                ok = cnt >= k
                nlos.append(jnp.where(ok, lo, mid))
                nhis.append(jnp.where(ok, mid, hi))
        return nlos, nhis

    los, his = jax.lax.fori_loop(0, 31, body, (los, his))
    return [lo if largest else hi
            for (keys, k, largest), lo, hi in zip(specs, los, his)]


def _weights_from_threshold(keys, t, k, largest, ut, ls):
    strict = (keys > t) if largest else (keys < t)
    cnt_strict = jnp.sum(strict.astype(jnp.int32))
    r = (jnp.int32(k) - cnt_strict).astype(jnp.float32)
    eq = keys == t
    rank = _flat_rank(eq, ut, ls)
    return strict | (eq & (rank <= r))


def _seg_a_body(vid_ref, ut_ref, ls_ref, c0_ref, wcode_ref):
    ut = ut_ref[...]
    ls = ls_ref[...]

    stds = (0.229, 0.224, 0.225)
    means = (0.485, 0.456, 0.406)

    vcs = []
    dsum = jnp.zeros((CROP, CROP), jnp.float32)
    dpair = [jnp.zeros((CROP, CROP), jnp.float32) for _ in range(8)]
    for c in range(3):
        tc = vid_ref[0, c] * stds[c] + means[c]
        vcs.append(jnp.mean(tc, axis=0))
        ad = jnp.abs(tc[:-1] - tc[1:])
        dsum = dsum + jnp.sum(ad, axis=0)
        for j in range(8):
            dpair[j] = dpair[j] + ad[2 * j]

    r, g, b = vcs
    maxc = jnp.maximum(jnp.maximum(r, g), b)
    minc = jnp.minimum(jnp.minimum(r, g), b)
    vv = maxc
    deltac = maxc - minc
    s = deltac / (maxc + 1e-08)
    dc = jnp.where(deltac == 0, 1.0, deltac)
    rc = (maxc - r) / dc
    gc = (maxc - g) / dc
    bc = (maxc - b) / dc
    h = jnp.where(maxc == r, bc - gc,
                  jnp.where(maxc == g, 2.0 + rc - bc, 4.0 + gc - rc))
    h6 = h / 6.0
    hmod = h6 - jnp.floor(h6)
    hang = hmod * (2.0 * np.pi)
    hx = (s * jnp.cos(hang * 2.0 * np.pi) + 1.0) / 2.0
    hy = (s * jnp.sin(hang * 2.0 * np.pi) + 1.0) / 2.0
    hq = jnp.round(hx * 9.0 + 1.0)
    sq = jnp.round(hy * 9.0 + 1.0)
    vq = jnp.round(vv * 9.0 + 1.0)
    cm = (hq + (sq - 1.0) * 10.0 + (vq - 1.0) * 100.0).astype(jnp.int32)
    c0 = cm - 1
    c0_ref[0] = c0

    maps = [dsum / 15.0] + dpair

    ms = [_blur_norm(maps[mi]) for mi in range(9)]
    keys = [jax.lax.bitcast_convert_type(m, jnp.int32) for m in ms]

    specs = []
    for mi in range(9):
        specs.append((keys[mi], K_FG, True))
        specs.append((keys[mi], K_BG, False))
    ths = _joint_bisect(specs)

    for mi in range(9):
        sfg = _weights_from_threshold(keys[mi], ths[2 * mi], K_FG, True, ut, ls)
        sbg = _weights_from_threshold(keys[mi], ths[2 * mi + 1], K_BG, False, ut, ls)
        wcode_ref[0, mi] = (sfg.astype(jnp.int32)
                            | (sbg.astype(jnp.int32) << 13))


def _seg_a(videos):
    B, C, T, H, W = videos.shape
    ut = jnp.asarray(_UT)
    ls = jnp.asarray(_LS)
    c2 = lambda bb: (0, 0)
    return pl.pallas_call(
        _seg_a_body,
        grid=(B,),
        in_specs=[
            pl.BlockSpec((1, C, T, H, W), lambda bb: (bb, 0, 0, 0, 0)),
            pl.BlockSpec((H, W), c2),
            pl.BlockSpec((H, W), c2),
        ],
        out_specs=[
            pl.BlockSpec((1, H, W), lambda bb: (bb, 0, 0)),
            pl.BlockSpec((1, 9, H, W), lambda bb: (bb, 0, 0, 0)),
        ],
        out_shape=[
            jax.ShapeDtypeStruct((B, H, W), jnp.int32),
            jax.ShapeDtypeStruct((B, 9, H, W), jnp.int32),
        ],
    )(videos, ut, ls)


_NBINP = 1008


def _sc_refine(c0f, wcf):
    B = c0f.shape[0] // HW
    mesh = plsc.VectorSubcoreMesh(core_axis_name="c", subcore_axis_name="s")

    @functools.partial(
        pl.kernel, mesh=mesh,
        out_type=jax.ShapeDtypeStruct((B * 9 * HW,), jnp.float32),
        compiler_params=pltpu.CompilerParams(needs_layout_passes=False),
        scratch_types=[
            pltpu.VMEM((HW,), jnp.int32),
            pltpu.VMEM((HW,), jnp.int32),
            pltpu.VMEM((16 * _NBINP,), jnp.int32),
            pltpu.VMEM((_NBINP,), jnp.float32),
            pltpu.VMEM((HW,), jnp.float32),
        ],
    )
    def k(c0_hbm, wc_hbm, out_hbm, c0_v, wc_v, h16_v, val_v, ref_v):
        wid = lax.axis_index("s") * 2 + lax.axis_index("c")
        pltpu.sync_copy(c0_hbm.at[pl.ds(wid * HW, HW)], c0_v)
        io16 = lax.broadcasted_iota(jnp.int32, (16,), 0)
        lane_off = io16 * _NBINP
        zz = jnp.zeros((16,), jnp.int32)
        for mi in range(9):
            pltpu.sync_copy(wc_hbm.at[pl.ds((wid * 9 + mi) * HW, HW)], wc_v)

            def zb(i, _):
                h16_v[pl.ds(i * 16, 16)] = zz
                return 0

            lax.fori_loop(0, _NBINP, zb, 0)

            def sb(i, _):
                idx = c0_v[pl.ds(i * 16, 16)]
                code = wc_v[pl.ds(i * 16, 16)]
                plsc.addupdate_scatter(h16_v, [idx + lane_off], code)
                return 0

            lax.fori_loop(0, HW // 16, sb, 0)

            def vb(i, _):
                acc = h16_v[pl.ds(i * 16, 16)]
                for l in range(1, 16):
                    acc = acc + h16_v[pl.ds(l * _NBINP + i * 16, 16)]
                hf = (acc & 0x1FFF).astype(jnp.float32)
                hb = (acc >> 13).astype(jnp.float32)
                df = hf / 6272.0
                db = (hb + 1.0) / 2254.0
                val_v[pl.ds(i * 16, 16)] = df / (df + db)
                return 0

            lax.fori_loop(0, _NBINP // 16, vb, 0)

            def gb(i, _):
                idx = c0_v[pl.ds(i * 16, 16)]
                ref_v[pl.ds(i * 16, 16)] = plsc.load_gather(val_v, [idx])
                return 0

            lax.fori_loop(0, HW // 16, gb, 0)
            pltpu.sync_copy(ref_v, out_hbm.at[pl.ds((wid * 9 + mi) * HW, HW)])

    return k(c0f, wcf)


def _seg_c_body(perm_ref, refine_ref, vidp_ref, vid_ref, ut_ref, ls_ref,
                pp_ref, av_ref, pools_ref):
    ut = ut_ref[...]
    ls = ls_ref[...]
    pp = pp_ref[...]

    m2s = [_blur_norm(refine_ref[0, mi]) for mi in range(9)]
    keys2 = [jax.lax.bitcast_convert_type(m2, jnp.int32) for m2 in m2s]
    ths2 = _joint_bisect([(k2, K_FINAL, True) for k2 in keys2])

    for mi in range(9):
        msk = _weights_from_threshold(
            keys2[mi], ths2[mi], K_FINAL, True, ut, ls).astype(jnp.float32)
        if mi == 0:
            av_ref[0] = vidp_ref[0] * (1.0 - msk) + vid_ref[0] * msk
        pool = jnp.dot(_dotT(pp, msk), pp,
                       preferred_element_type=jnp.float32) / 256.0
        pools_ref[0, mi, 0:7, 0:7] = pool


def _seg_c(refine, videos, perm):
    B, C, T, H, W = videos.shape
    ut = jnp.asarray(_UT)
    ls = jnp.asarray(_LS)
    pp = jnp.asarray(_PP)
    c2 = lambda bb, p: (0, 0)
    grid_spec = pltpu.PrefetchScalarGridSpec(
        num_scalar_prefetch=1,
        grid=(B,),
        in_specs=[
            pl.BlockSpec((1, 9, H, W), lambda bb, p: (bb, 0, 0, 0)),
            pl.BlockSpec((1, C, T, H, W), lambda bb, p: (p[bb], 0, 0, 0, 0)),
            pl.BlockSpec((1, C, T, H, W), lambda bb, p: (bb, 0, 0, 0, 0)),
            pl.BlockSpec((H, W), c2),
            pl.BlockSpec((H, W), c2),
            pl.BlockSpec((H, 7), c2),
        ],
        out_specs=[
            pl.BlockSpec((1, C, T, H, W), lambda bb, p: (bb, 0, 0, 0, 0)),
            pl.BlockSpec((1, 9, 8, 128), lambda bb, p: (bb, 0, 0, 0)),
        ],
    )
    return pl.pallas_call(
        _seg_c_body,
        grid_spec=grid_spec,
        out_shape=[
            jax.ShapeDtypeStruct((B, C, T, H, W), jnp.float32),
            jax.ShapeDtypeStruct((B, 9, 8, 128), jnp.float32),
        ],
    )(perm, refine, videos, videos, ut, ls, pp)


def _fuse_body(perm_ref, vidp_ref, vid_ref, mask_ref, out_ref):
    m = mask_ref[0]
    out_ref[0] = vidp_ref[0] * (1.0 - m) + vid_ref[0] * m


def _fuse(videos, mask, perm):
    B, C, T, H, W = videos.shape
    grid_spec = pltpu.PrefetchScalarGridSpec(
        num_scalar_prefetch=1,
        grid=(B,),
        in_specs=[
            pl.BlockSpec((1, C, T, H, W), lambda b, p: (p[b], 0, 0, 0, 0)),
            pl.BlockSpec((1, C, T, H, W), lambda b, p: (b, 0, 0, 0, 0)),
            pl.BlockSpec((1, H, W), lambda b, p: (b, 0, 0)),
        ],
        out_specs=pl.BlockSpec((1, C, T, H, W), lambda b, p: (b, 0, 0, 0, 0)),
    )
    return pl.pallas_call(
        _fuse_body,
        grid_spec=grid_spec,
        out_shape=jax.ShapeDtypeStruct((B, C, T, H, W), videos.dtype),
    )(perm, videos, videos, mask)


def kernel(videos, label):
    B, C, T, H, W = videos.shape
    c0, wcode = _seg_a(videos)
    refine = _sc_refine(c0.reshape(B * HW), wcode.reshape(B * 9 * HW))
    index = jax.random.permutation(jax.random.key(1234), B).astype(jnp.int32)
    all_videos, pools = _seg_c(refine.reshape(B, 9, H, W), videos, index)
    mask_out = pools[:, 0, :7, :7].reshape(B, 49)
    mpf_out = pools[:, 1:9, :7, :7].reshape(B, 392)
    return (all_videos, label, (mask_out, mpf_out))


# speedup vs baseline: 74.0359x; 1.0383x over previous
import functools

import jax
import jax.numpy as jnp
import numpy as np
from jax import lax
from jax.experimental import pallas as pl
from jax.experimental.pallas import tpu as pltpu
from jax.experimental.pallas import tpu_sc as plsc

CROP = 112
BETA = 0.5
EPS = 1e-08
GK = (int(0.1 * CROP) // 2) * 2 + 1
SIGMA = GK / 3.0
HW = CROP * CROP
K_FG = int(0.5 * HW)
K_BG = int(0.1 * HW)
K_FINAL = int(BETA * HW)
NBINS = 1000


def _np_blur_matrix():
    x = np.arange(GK, dtype=np.float64) - GK // 2
    g = np.exp(-(x ** 2) / (2.0 * SIGMA ** 2))
    g = (g / g.sum()).astype(np.float32)
    n = CROP
    KB = np.zeros((n, n), dtype=np.float32)
    for i in range(n):
        for k in range(GK):
            src = i - GK // 2 + k
            if src < 0:
                src = -src
            elif src > n - 1:
                src = 2 * (n - 1) - src
            KB[i, src] += g[k]
    return KB


_KB = _np_blur_matrix()
_UT = np.triu(np.ones((CROP, CROP), dtype=np.float32))
_LS = np.tril(np.ones((CROP, CROP), dtype=np.float32), k=-1)
_PP = np.zeros((CROP, 7), dtype=np.float32)
for _i in range(CROP):
    _PP[_i, _i // 16] = 1.0


def _dotT(a, b):
    return jax.lax.dot_general(a, b, (((0,), (0,)), ((), ())),
                               preferred_element_type=jnp.float32)


def _gauss_np():
    x = np.arange(GK, dtype=np.float32) - GK // 2
    g = np.exp(-(x ** 2) / (2.0 * SIGMA ** 2)).astype(np.float32)
    g = (g / g.sum()).astype(np.float32)
    return np.asarray(g, dtype=np.float32).astype("bfloat16").astype(np.float32)


_GB = [float(w) for w in _gauss_np()]


def _bf16(x):
    return x.astype(jnp.bfloat16).astype(jnp.float32)


def _blur_norm(x):
    top = [x[5 - p:6 - p, :] for p in range(5)]
    bot = [x[110 - q:111 - q, :] for q in range(5)]
    xp = _bf16(jnp.concatenate(top + [x] + bot, axis=0))
    y = _GB[0] * xp[0:CROP, :]
    for k in range(1, GK):
        y = y + _GB[k] * xp[k:k + CROP, :]
    left = [y[:, 5 - p:6 - p] for p in range(5)]
    right = [y[:, 110 - q:111 - q] for q in range(5)]
    yp = _bf16(jnp.concatenate(left + [y] + right, axis=1))
    z = _GB[0] * yp[:, 0:CROP]
    for k in range(1, GK):
        z = z + _GB[k] * yp[:, k:k + CROP]
    mn = jnp.min(jnp.min(z, axis=1, keepdims=True), axis=0, keepdims=True)
    f = z - mn
    mx = jnp.max(jnp.max(f, axis=1, keepdims=True), axis=0, keepdims=True)
    return f / (mx + EPS)


def _flat_rank(eq, ut, ls):
    eqf = eq.astype(jnp.float32)
    rowcum = jnp.dot(eqf, ut, preferred_element_type=jnp.float32)
    rowtot = rowcum[:, CROP - 1:CROP]
    offs = jnp.dot(ls, rowtot, preferred_element_type=jnp.float32)
    return rowcum + offs


_TOPBITS = 0x3F800000


def _joint_bisect(specs):
    los, his = [], []
    for keys, k, largest in specs:
        if largest:
            los.append(jnp.int32(0))
            his.append(jnp.int32(_TOPBITS))
        else:
            los.append(jnp.int32(-1))
            his.append(jnp.int32(_TOPBITS))

    def body(_, carry):
        los, his = carry
        nlos, nhis = [], []
        for (keys, k, largest), lo, hi in zip(specs, los, his):
            mid = (lo + hi) >> 1
            if largest:
                cnt = jnp.sum((keys >= mid).astype(jnp.int32))
                ok = cnt >= k
                nlos.append(jnp.where(ok, mid, lo))
                nhis.append(jnp.where(ok, hi, mid))
            else:
                cnt = jnp.sum((keys <= mid).astype(jnp.int32))
                ok = cnt >= k
                nlos.append(jnp.where(ok, lo, mid))
                nhis.append(jnp.where(ok, mid, hi))
        return nlos, nhis

    los, his = jax.lax.fori_loop(0, 31, body, (los, his))
    return [lo if largest else hi
            for (keys, k, largest), lo, hi in zip(specs, los, his)]


def _weights_from_threshold(keys, t, k, largest, ut, ls):
    strict = (keys > t) if largest else (keys < t)
    cnt_strict = jnp.sum(strict.astype(jnp.int32))
    r = (jnp.int32(k) - cnt_strict).astype(jnp.float32)
    eq = keys == t
    rank = _flat_rank(eq, ut, ls)
    return strict | (eq & (rank <= r))


def _seg_a_body(vid_ref, ut_ref, ls_ref, c0_ref, wcode_ref):
    ut = ut_ref[...]
    ls = ls_ref[...]

    stds = (0.229, 0.224, 0.225)
    means = (0.485, 0.456, 0.406)

    vcs = []
    dsum = jnp.zeros((CROP, CROP), jnp.float32)
    dpair = [jnp.zeros((CROP, CROP), jnp.float32) for _ in range(8)]
    for c in range(3):
        tc = vid_ref[0, c] * stds[c] + means[c]
        vcs.append(jnp.mean(tc, axis=0))
        ad = jnp.abs(tc[:-1] - tc[1:])
        dsum = dsum + jnp.sum(ad, axis=0)
        for j in range(8):
            dpair[j] = dpair[j] + ad[2 * j]

    r, g, b = vcs
    maxc = jnp.maximum(jnp.maximum(r, g), b)
    minc = jnp.minimum(jnp.minimum(r, g), b)
    vv = maxc
    deltac = maxc - minc
    s = deltac / (maxc + 1e-08)
    dc = jnp.where(deltac == 0, 1.0, deltac)
    rc = (maxc - r) / dc
    gc = (maxc - g) / dc
    bc = (maxc - b) / dc
    h = jnp.where(maxc == r, bc - gc,
                  jnp.where(maxc == g, 2.0 + rc - bc, 4.0 + gc - rc))
    h6 = h / 6.0
    hmod = h6 - jnp.floor(h6)
    hang = hmod * (2.0 * np.pi)
    hx = (s * jnp.cos(hang * 2.0 * np.pi) + 1.0) / 2.0
    hy = (s * jnp.sin(hang * 2.0 * np.pi) + 1.0) / 2.0
    hq = jnp.round(hx * 9.0 + 1.0)
    sq = jnp.round(hy * 9.0 + 1.0)
    vq = jnp.round(vv * 9.0 + 1.0)
    cm = (hq + (sq - 1.0) * 10.0 + (vq - 1.0) * 100.0).astype(jnp.int32)
    c0 = cm - 1
    c0_ref[0] = c0

    maps = [dsum / 15.0] + dpair

    ms = [_blur_norm(maps[mi]) for mi in range(9)]
    keys = [jax.lax.bitcast_convert_type(m, jnp.int32) for m in ms]

    specs = []
    for mi in range(9):
        specs.append((keys[mi], K_FG, True))
        specs.append((keys[mi], K_BG, False))
    ths = _joint_bisect(specs)

    for mi in range(9):
        sfg = _weights_from_threshold(keys[mi], ths[2 * mi], K_FG, True, ut, ls)
        sbg = _weights_from_threshold(keys[mi], ths[2 * mi + 1], K_BG, False, ut, ls)
        wcode_ref[0, mi] = (sfg.astype(jnp.int32)
                            | (sbg.astype(jnp.int32) << 13))


def _seg_a(videos):
    B, C, T, H, W = videos.shape
    ut = jnp.asarray(_UT)
    ls = jnp.asarray(_LS)
    c2 = lambda bb: (0, 0)
    return pl.pallas_call(
        _seg_a_body,
        grid=(B,),
        in_specs=[
            pl.BlockSpec((1, C, T, H, W), lambda bb: (bb, 0, 0, 0, 0)),
            pl.BlockSpec((H, W), c2),
            pl.BlockSpec((H, W), c2),
        ],
        out_specs=[
            pl.BlockSpec((1, H, W), lambda bb: (bb, 0, 0)),
            pl.BlockSpec((1, 9, H, W), lambda bb: (bb, 0, 0, 0)),
        ],
        out_shape=[
            jax.ShapeDtypeStruct((B, H, W), jnp.int32),
            jax.ShapeDtypeStruct((B, 9, H, W), jnp.int32),
        ],
    )(videos, ut, ls)


_NBINP = 1008


def _sc_refine(c0f, wcf):
    B = c0f.shape[0] // HW
    mesh = plsc.VectorSubcoreMesh(core_axis_name="c", subcore_axis_name="s")

    @functools.partial(
        pl.kernel, mesh=mesh,
        out_type=jax.ShapeDtypeStruct((B * 9 * HW,), jnp.float32),
        compiler_params=pltpu.CompilerParams(needs_layout_passes=False),
        scratch_types=[
            pltpu.VMEM((HW,), jnp.int32),
            pltpu.VMEM((HW,), jnp.int32),
            pltpu.VMEM((16 * _NBINP,), jnp.int32),
            pltpu.VMEM((_NBINP,), jnp.float32),
            pltpu.VMEM((HW,), jnp.float32),
        ],
    )
    def k(c0_hbm, wc_hbm, out_hbm, c0_v, wc_v, h16_v, val_v, ref_v):
        wid = lax.axis_index("s") * 2 + lax.axis_index("c")
        pltpu.sync_copy(c0_hbm.at[pl.ds(wid * HW, HW)], c0_v)
        io16 = lax.broadcasted_iota(jnp.int32, (16,), 0)
        lane_off = io16 * _NBINP
        zz = jnp.zeros((16,), jnp.int32)
        for mi in range(9):
            pltpu.sync_copy(wc_hbm.at[pl.ds((wid * 9 + mi) * HW, HW)], wc_v)

            def zb(i, _):
                for u in range(4):
                    h16_v[pl.ds(i * 64 + u * 16, 16)] = zz
                return 0

            lax.fori_loop(0, _NBINP // 4, zb, 0)

            def sb(i, _):
                for u in range(4):
                    o = i * 64 + u * 16
                    idx = c0_v[pl.ds(o, 16)]
                    code = wc_v[pl.ds(o, 16)]
                    plsc.addupdate_scatter(h16_v, [idx + lane_off], code)
                return 0

            lax.fori_loop(0, HW // 64, sb, 0)

            def vb(i, _):
                acc = h16_v[pl.ds(i * 16, 16)]
                for l in range(1, 16):
                    acc = acc + h16_v[pl.ds(l * _NBINP + i * 16, 16)]
                hf = (acc & 0x1FFF).astype(jnp.float32)
                hb = (acc >> 13).astype(jnp.float32)
                df = hf / 6272.0
                db = (hb + 1.0) / 2254.0
                val_v[pl.ds(i * 16, 16)] = df / (df + db)
                return 0

            lax.fori_loop(0, _NBINP // 16, vb, 0)

            def gb(i, _):
                for u in range(4):
                    o = i * 64 + u * 16
                    idx = c0_v[pl.ds(o, 16)]
                    ref_v[pl.ds(o, 16)] = plsc.load_gather(val_v, [idx])
                return 0

            lax.fori_loop(0, HW // 64, gb, 0)
            pltpu.sync_copy(ref_v, out_hbm.at[pl.ds((wid * 9 + mi) * HW, HW)])

    return k(c0f, wcf)


def _seg_c_body(perm_ref, refine_ref, vidp_ref, vid_ref, ut_ref, ls_ref,
                pp_ref, av_ref, pools_ref):
    ut = ut_ref[...]
    ls = ls_ref[...]
    pp = pp_ref[...]

    m2s = [_blur_norm(refine_ref[0, mi]) for mi in range(9)]
    keys2 = [jax.lax.bitcast_convert_type(m2, jnp.int32) for m2 in m2s]
    ths2 = _joint_bisect([(k2, K_FINAL, True) for k2 in keys2])

    for mi in range(9):
        msk = _weights_from_threshold(
            keys2[mi], ths2[mi], K_FINAL, True, ut, ls).astype(jnp.float32)
        if mi == 0:
            av_ref[0] = vidp_ref[0] * (1.0 - msk) + vid_ref[0] * msk
        pool = jnp.dot(_dotT(pp, msk), pp,
                       preferred_element_type=jnp.float32) / 256.0
        pools_ref[0, mi, 0:7, 0:7] = pool


def _seg_c(refine, videos, perm):
    B, C, T, H, W = videos.shape
    ut = jnp.asarray(_UT)
    ls = jnp.asarray(_LS)
    pp = jnp.asarray(_PP)
    c2 = lambda bb, p: (0, 0)
    grid_spec = pltpu.PrefetchScalarGridSpec(
        num_scalar_prefetch=1,
        grid=(B,),
        in_specs=[
            pl.BlockSpec((1, 9, H, W), lambda bb, p: (bb, 0, 0, 0)),
            pl.BlockSpec((1, C, T, H, W), lambda bb, p: (p[bb], 0, 0, 0, 0)),
            pl.BlockSpec((1, C, T, H, W), lambda bb, p: (bb, 0, 0, 0, 0)),
            pl.BlockSpec((H, W), c2),
            pl.BlockSpec((H, W), c2),
            pl.BlockSpec((H, 7), c2),
        ],
        out_specs=[
            pl.BlockSpec((1, C, T, H, W), lambda bb, p: (bb, 0, 0, 0, 0)),
            pl.BlockSpec((1, 9, 8, 128), lambda bb, p: (bb, 0, 0, 0)),
        ],
    )
    return pl.pallas_call(
        _seg_c_body,
        grid_spec=grid_spec,
        out_shape=[
            jax.ShapeDtypeStruct((B, C, T, H, W), jnp.float32),
            jax.ShapeDtypeStruct((B, 9, 8, 128), jnp.float32),
        ],
    )(perm, refine, videos, videos, ut, ls, pp)


def _fuse_body(perm_ref, vidp_ref, vid_ref, mask_ref, out_ref):
    m = mask_ref[0]
    out_ref[0] = vidp_ref[0] * (1.0 - m) + vid_ref[0] * m


def _fuse(videos, mask, perm):
    B, C, T, H, W = videos.shape
    grid_spec = pltpu.PrefetchScalarGridSpec(
        num_scalar_prefetch=1,
        grid=(B,),
        in_specs=[
            pl.BlockSpec((1, C, T, H, W), lambda b, p: (p[b], 0, 0, 0, 0)),
            pl.BlockSpec((1, C, T, H, W), lambda b, p: (b, 0, 0, 0, 0)),
            pl.BlockSpec((1, H, W), lambda b, p: (b, 0, 0)),
        ],
        out_specs=pl.BlockSpec((1, C, T, H, W), lambda b, p: (b, 0, 0, 0, 0)),
    )
    return pl.pallas_call(
        _fuse_body,
        grid_spec=grid_spec,
        out_shape=jax.ShapeDtypeStruct((B, C, T, H, W), videos.dtype),
    )(perm, videos, videos, mask)


def kernel(videos, label):
    B, C, T, H, W = videos.shape
    c0, wcode = _seg_a(videos)
    refine = _sc_refine(c0.reshape(B * HW), wcode.reshape(B * 9 * HW))
    index = jax.random.permutation(jax.random.key(1234), B).astype(jnp.int32)
    all_videos, pools = _seg_c(refine.reshape(B, 9, H, W), videos, index)
    mask_out = pools[:, 0, :7, :7].reshape(B, 49)
    mpf_out = pools[:, 1:9, :7, :7].reshape(B, 392)
    return (all_videos, label, (mask_out, mpf_out))
